# baseline (speedup 1.0000x reference)
import functools

import jax
import jax.numpy as jnp
from jax import lax
from jax.experimental import pallas as pl
from jax.experimental.pallas import tpu as pltpu
from jax.experimental.pallas import tpu_sc as plsc

_N = 50000
_E = 800000
_C = 16
_K = 64
_NCHUNK = _E // _K
_MAIN = (_NCHUNK // 16) // 6 * 6
_EXTRA = _NCHUNK - 16 * (_MAIN + 1)
_ACC_ROWS = 50048
_RPT = _ACC_ROWS // 16

_mesh = plsc.VectorSubcoreMesh(core_axis_name="c", subcore_axis_name="s")


@functools.partial(
    pl.kernel,
    out_type=jax.ShapeDtypeStruct((2 * _ACC_ROWS, 32), jnp.float32),
    mesh=_mesh,
    compiler_params=pltpu.CompilerParams(needs_layout_passes=False,
                                         use_tc_tiling_on_sc=False),
    scratch_types=[
        [pltpu.VMEM((_K,), jnp.int32)] * 3,
        [pltpu.VMEM((_K,), jnp.int32)] * 3,
        [pltpu.VMEM((16, _K), jnp.float32)] * 3,
        [pltpu.VMEM((48, _K), jnp.float32)] * 3,
        [pltpu.VMEM((_K, 64), jnp.float32)] * 2,
        [pltpu.VMEM((_K, 32), jnp.float32)] * 2,
        [pltpu.VMEM((_K,), jnp.int32)] * 2,
        pltpu.VMEM_SHARED((_ACC_ROWS, 32), jnp.float32),
        [pltpu.SemaphoreType.DMA] * 3,
        [pltpu.SemaphoreType.DMA] * 2,
        [pltpu.SemaphoreType.DMA] * 2,
    ],
)
def _edge_kernel(htab, srcm, nbrm, ea0m, ea1m, out_hbm,
                 nbr, src, ea0, ea1, g, o, idx, acc,
                 semlin, semg, semsc):
    c = lax.axis_index("c")
    s = lax.axis_index("s")
    nt = jnp.where(s < _EXTRA, _MAIN + 2, _MAIN + 1)

    cm = (jnp.zeros((_C,), jnp.int32) + c) == 0
    zeros16 = jnp.zeros((_C,), jnp.float32)
    iota = lax.iota(jnp.int32, _C)

    def _lin_refs(j, r):
        return (
            (nbrm.at[pl.ds(j * _K, _K)], nbr[r]),
            (srcm.at[pl.ds(j * _K, _K)], src[r]),
            (ea0m.at[:, pl.ds(j * _K, _K)], ea0[r]),
            (ea1m.at[:, pl.ds(j * _K, _K)], ea1[r]),
        )

    def _issue_lin(j, r):
        for a, b in _lin_refs(j, r):
            pltpu.async_copy(a, b, semlin[r])

    def _wait_lin(j, r):
        for a, b in _lin_refs(j, r):
            pltpu.make_async_copy(a, b, semlin[r]).wait()

    def _wait_scatter(p):
        pltpu.make_async_copy(o[p], acc.at[idx[p]], semsc[p]).wait()

    _issue_lin(s, 0)
    _issue_lin(s + 16, 1)

    @plsc.parallel_loop(0, _K)
    def _zero_row(e):
        o[0][e, pl.ds(0, 16)] = zeros16
        o[0][e, pl.ds(16, 16)] = zeros16

    row0 = s * _RPT
    for k in range(_RPT // _K):
        pltpu.sync_copy(o[0], acc.at[pl.ds(row0 + k * _K, _K)])
    pltpu.sync_copy(o[0].at[pl.ds(0, _RPT % _K)],
                    acc.at[pl.ds(row0 + _RPT // _K * _K, _RPT % _K)])

    _wait_lin(s, 0)
    pltpu.async_copy(htab.at[nbr[0]], g[0], semg[0])
    plsc.subcore_barrier()

    def body(n, k):
        r, r1, r2 = k % 3, (k + 1) % 3, (k + 2) % 3
        p, p1 = k % 2, (k + 1) % 2
        j1 = s + (n + 1) * 16
        j2 = s + (n + 2) * 16

        @pl.when(n + 1 < nt)
        def _():
            _wait_lin(j1, r1)
            pltpu.async_copy(htab.at[nbr[r1]], g[p1], semg[p1])

        @pl.when(n + 2 < nt)
        def _():
            _issue_lin(j2, r2)

        pltpu.make_async_copy(htab.at[nbr[r]], g[p], semg[p]).wait()

        @pl.when(n >= 2)
        def _():
            _wait_scatter(p)

        @plsc.parallel_loop(0, _K // _C)
        def _idx_copy(i):
            idx[p][pl.ds(i * _C, _C)] = src[r][pl.ds(i * _C, _C)]

        gv, e0v, e1v, ov = g[p], ea0[r], ea1[r], o[p]

        @plsc.parallel_loop(0, _K, unroll=2)
        def _edge(e):
            g0 = gv[e, pl.ds(0, 16)]
            g1x = gv[e, pl.ds(16, 16)]
            g1y = gv[e, pl.ds(32, 16)]
            g1z = gv[e, pl.ds(48, 16)]
            es = jnp.full((_C,), e, jnp.int32)
            a0 = plsc.load_gather(e0v, [iota, es])
            a1x = plsc.load_gather(e1v, [iota, es])
            a1y = plsc.load_gather(e1v, [iota + 16, es])
            a1z = plsc.load_gather(e1v, [iota + 32, es])
            o0 = g0 * a0 + g1x * a1x + g1y * a1y + g1z * a1z
            px = g0 * a1x + g1x * a0
            py = g0 * a1y + g1y * a0
            pz = g0 * a1z + g1z * a0
            ov[e, pl.ds(0, 16)] = jnp.where(cm, o0, py)
            ov[e, pl.ds(16, 16)] = jnp.where(cm, px, pz)

        pltpu.async_copy(ov, acc.at[idx[p]], semsc[p], add=True)

    def outer(m, _):
        for k in range(6):
            body(m * 6 + k, k)
        return 0

    lax.fori_loop(0, _MAIN // 6, outer, 0)

    body(_MAIN, 0)

    @pl.when(_MAIN + 1 < nt)
    def _():
        body(_MAIN + 1, 1)

    _wait_scatter(0)
    _wait_scatter(1)
    plsc.subcore_barrier()

    pltpu.sync_copy(acc.at[pl.ds(row0, _RPT)],
                    out_hbm.at[pl.ds(c * _ACC_ROWS + row0, _RPT)])


def kernel(h_0, h_1, rel_pos, edge_index, edge_attr_0, edge_attr_1,
           channel_weights):
    del rel_pos, channel_weights
    n = h_0.shape[0]
    htab = jnp.concatenate(
        [h_0, h_1[:, :, 0], h_1[:, :, 1], h_1[:, :, 2]], axis=1)
    src = edge_index[0].astype(jnp.int32)
    nbr = edge_index[1].astype(jnp.int32)
    ea0_p = edge_attr_0.T
    ea1_p = edge_attr_1.transpose(2, 1, 0).reshape(48, _E)
    out = _edge_kernel(htab, src, nbr, ea0_p,
                       ea1_p).reshape(2, _ACC_ROWS, 32)
    a, b = out[0, :n], out[1, :n]
    out0 = a[:, :16]
    out1 = jnp.stack([a[:, 16:], b[:, :16], b[:, 16:]], axis=-1)
    return (out0, out1)


# speedup vs baseline: 2.7589x; 2.7589x over previous
import functools

import jax
import jax.numpy as jnp
from jax import lax
from jax.experimental import pallas as pl
from jax.experimental.pallas import tpu as pltpu
from jax.experimental.pallas import tpu_sc as plsc

_N = 50000
_E = 800000
_C = 16
_K = 64
_NCHUNK = _E // _K
_MAIN = (_NCHUNK // 16) // 6 * 6
_EXTRA = _NCHUNK - 16 * (_MAIN + 1)
_ACC_ROWS = 50048
_RPT = _ACC_ROWS // 16

_mesh = plsc.VectorSubcoreMesh(core_axis_name="c", subcore_axis_name="s")


@functools.partial(
    pl.kernel,
    out_type=jax.ShapeDtypeStruct((2 * _ACC_ROWS, 32), jnp.float32),
    mesh=_mesh,
    compiler_params=pltpu.CompilerParams(needs_layout_passes=False,
                                         use_tc_tiling_on_sc=False),
    scratch_types=[
        [pltpu.VMEM((_K,), jnp.int32)] * 3,
        [pltpu.VMEM((_K,), jnp.int32)] * 3,
        [pltpu.VMEM((2, 1, 8, _K), jnp.float32)] * 3,
        [pltpu.VMEM((3, 2, 1, 8, _K), jnp.float32)] * 3,
        [pltpu.VMEM((_K, 64), jnp.float32)] * 2,
        [pltpu.VMEM((_K, 32), jnp.float32)] * 2,
        [pltpu.VMEM((_K,), jnp.int32)] * 2,
        pltpu.VMEM_SHARED((_ACC_ROWS, 32), jnp.float32),
        [pltpu.SemaphoreType.DMA] * 3,
        [pltpu.SemaphoreType.DMA] * 2,
        [pltpu.SemaphoreType.DMA] * 2,
    ],
)
def _edge_kernel(htab, srcm, nbrm, ea0m, ea1m, out_hbm,
                 nbr, src, ea0, ea1, g, o, idx, acc,
                 semlin, semg, semsc):
    c = lax.axis_index("c")
    s = lax.axis_index("s")
    nt = jnp.where(s < _EXTRA, _MAIN + 2, _MAIN + 1)

    cm = (jnp.zeros((_C,), jnp.int32) + c) == 0
    zeros16 = jnp.zeros((_C,), jnp.float32)
    iota = lax.iota(jnp.int32, _C)
    chb_v = iota // 8
    ch8_v = iota - chb_v * 8
    z_v = jnp.zeros((_C,), jnp.int32)
    d1_v = z_v + 1
    d2_v = z_v + 2

    def _lin_refs(j, r):
        eb, half = j // 2, (j % 2) * _K
        return (
            (nbrm.at[pl.ds(j * _K, _K)], nbr[r]),
            (srcm.at[pl.ds(j * _K, _K)], src[r]),
            (ea0m.at[:, pl.ds(eb, 1), :, pl.ds(half, _K)], ea0[r]),
            (ea1m.at[:, :, pl.ds(eb, 1), :, pl.ds(half, _K)], ea1[r]),
        )

    def _issue_lin(j, r):
        for a, b in _lin_refs(j, r):
            pltpu.async_copy(a, b, semlin[r])

    def _wait_lin(j, r):
        for a, b in _lin_refs(j, r):
            pltpu.make_async_copy(a, b, semlin[r]).wait()

    def _wait_scatter(p):
        pltpu.make_async_copy(o[p], acc.at[idx[p]], semsc[p]).wait()

    _issue_lin(s, 0)
    _issue_lin(s + 16, 1)

    @plsc.parallel_loop(0, _K)
    def _zero_row(e):
        o[0][e, pl.ds(0, 16)] = zeros16
        o[0][e, pl.ds(16, 16)] = zeros16

    row0 = s * _RPT
    for k in range(_RPT // _K):
        pltpu.sync_copy(o[0], acc.at[pl.ds(row0 + k * _K, _K)])
    pltpu.sync_copy(o[0].at[pl.ds(0, _RPT % _K)],
                    acc.at[pl.ds(row0 + _RPT // _K * _K, _RPT % _K)])

    _wait_lin(s, 0)
    pltpu.async_copy(htab.at[nbr[0]], g[0], semg[0])
    plsc.subcore_barrier()

    def body(n, k):
        r, r1, r2 = k % 3, (k + 1) % 3, (k + 2) % 3
        p, p1 = k % 2, (k + 1) % 2
        j1 = s + (n + 1) * 16
        j2 = s + (n + 2) * 16

        @pl.when(n + 1 < nt)
        def _():
            _wait_lin(j1, r1)
            pltpu.async_copy(htab.at[nbr[r1]], g[p1], semg[p1])

        @pl.when(n + 2 < nt)
        def _():
            _issue_lin(j2, r2)

        pltpu.make_async_copy(htab.at[nbr[r]], g[p], semg[p]).wait()

        @pl.when(n >= 2)
        def _():
            _wait_scatter(p)

        @plsc.parallel_loop(0, _K // _C)
        def _idx_copy(i):
            idx[p][pl.ds(i * _C, _C)] = src[r][pl.ds(i * _C, _C)]

        gv, e0v, e1v, ov = g[p], ea0[r], ea1[r], o[p]

        @plsc.parallel_loop(0, _K, unroll=2)
        def _edge(e):
            g0 = gv[e, pl.ds(0, 16)]
            g1x = gv[e, pl.ds(16, 16)]
            g1y = gv[e, pl.ds(32, 16)]
            g1z = gv[e, pl.ds(48, 16)]
            es = jnp.full((_C,), e, jnp.int32)
            a0 = plsc.load_gather(e0v, [chb_v, z_v, ch8_v, es])
            a1x = plsc.load_gather(e1v, [z_v, chb_v, z_v, ch8_v, es])
            a1y = plsc.load_gather(e1v, [d1_v, chb_v, z_v, ch8_v, es])
            a1z = plsc.load_gather(e1v, [d2_v, chb_v, z_v, ch8_v, es])
            o0 = g0 * a0 + g1x * a1x + g1y * a1y + g1z * a1z
            px = g0 * a1x + g1x * a0
            py = g0 * a1y + g1y * a0
            pz = g0 * a1z + g1z * a0
            ov[e, pl.ds(0, 16)] = jnp.where(cm, o0, py)
            ov[e, pl.ds(16, 16)] = jnp.where(cm, px, pz)

        pltpu.async_copy(ov, acc.at[idx[p]], semsc[p], add=True)

    def outer(m, _):
        for k in range(6):
            body(m * 6 + k, k)
        return 0

    lax.fori_loop(0, _MAIN // 6, outer, 0)

    body(_MAIN, 0)

    @pl.when(_MAIN + 1 < nt)
    def _():
        body(_MAIN + 1, 1)

    _wait_scatter(0)
    _wait_scatter(1)
    plsc.subcore_barrier()

    pltpu.sync_copy(acc.at[pl.ds(row0, _RPT)],
                    out_hbm.at[pl.ds(c * _ACC_ROWS + row0, _RPT)])


def kernel(h_0, h_1, rel_pos, edge_index, edge_attr_0, edge_attr_1,
           channel_weights):
    del rel_pos, channel_weights
    n = h_0.shape[0]
    nb = _NCHUNK // 2
    htab = jnp.concatenate(
        [h_0, h_1[:, :, 0], h_1[:, :, 1], h_1[:, :, 2],
         jnp.zeros((n, 64), jnp.float32)], axis=1).reshape(2 * n, 64)
    src = edge_index[0].astype(jnp.int32)
    nbr2 = edge_index[1].astype(jnp.int32) * 2
    ea0_t = (edge_attr_0.T.reshape(2, 8, nb, 128)
             .transpose(0, 2, 1, 3))
    ea1_t = (edge_attr_1.transpose(2, 1, 0).reshape(3, 2, 8, nb, 128)
             .transpose(0, 1, 3, 2, 4))
    out = _edge_kernel(htab, src, nbr2, ea0_t,
                       ea1_t).reshape(2, _ACC_ROWS, 32)
    a, b = out[0, :n], out[1, :n]
    out0 = a[:, :16]
    out1 = jnp.stack([a[:, 16:], b[:, :16], b[:, 16:]], axis=-1)
    return (out0, out1)
